# baseline (speedup 1.0000x reference)
import jax
import jax.numpy as jnp
from jax.experimental import pallas as pl
from jax.experimental.pallas import tpu as pltpu

_NBUF = 8
_K = _NBUF // 2
_CHUNK_B = 32


def _dma_pipe(x_ref, o_ref, buf, in_sems, out_sems):
    C = x_ref.shape[0] // _CHUNK_B

    def in_copy(i):
        s = i % _NBUF
        return pltpu.make_async_copy(
            x_ref.at[pl.ds(i * _CHUNK_B, _CHUNK_B)], buf.at[s], in_sems.at[s])

    def out_copy(i):
        s = i % _NBUF
        return pltpu.make_async_copy(
            buf.at[s], o_ref.at[pl.ds(i * _CHUNK_B, _CHUNK_B)], out_sems.at[s])

    waited_outs = set()
    for j in range(min(_K, C)):
        in_copy(j).start()
    for i in range(C):
        j = i + _K
        if j < C:
            if j - _NBUF >= 0:
                out_copy(j - _NBUF).wait()
                waited_outs.add(j - _NBUF)
            in_copy(j).start()
        in_copy(i).wait()
        out_copy(i).start()
    for i in range(C):
        if i not in waited_outs:
            out_copy(i).wait()


def kernel(x, hippocampus, neocortex):
    B, S, H = x.shape
    return pl.pallas_call(
        _dma_pipe,
        out_shape=jax.ShapeDtypeStruct(x.shape, x.dtype),
        in_specs=[pl.BlockSpec(memory_space=pl.ANY)],
        out_specs=pl.BlockSpec(memory_space=pl.ANY),
        scratch_shapes=[
            pltpu.VMEM((_NBUF, _CHUNK_B, S, H), x.dtype),
            pltpu.SemaphoreType.DMA((_NBUF,)),
            pltpu.SemaphoreType.DMA((_NBUF,)),
        ],
    )(x)


# speedup vs baseline: 1.0016x; 1.0016x over previous
import jax
import jax.numpy as jnp
from jax.experimental import pallas as pl
from jax.experimental.pallas import tpu as pltpu

_NBUF = 4
_K = _NBUF // 2
_CHUNK_B = 128


def _dma_pipe(x_ref, o_ref, buf, in_sems, out_sems):
    C = x_ref.shape[0] // _CHUNK_B

    def in_copy(i):
        s = i % _NBUF
        return pltpu.make_async_copy(
            x_ref.at[pl.ds(i * _CHUNK_B, _CHUNK_B)], buf.at[s], in_sems.at[s])

    def out_copy(i):
        s = i % _NBUF
        return pltpu.make_async_copy(
            buf.at[s], o_ref.at[pl.ds(i * _CHUNK_B, _CHUNK_B)], out_sems.at[s])

    waited_outs = set()
    for j in range(min(_K, C)):
        in_copy(j).start()
    for i in range(C):
        j = i + _K
        if j < C:
            if j - _NBUF >= 0:
                out_copy(j - _NBUF).wait()
                waited_outs.add(j - _NBUF)
            in_copy(j).start()
        in_copy(i).wait()
        out_copy(i).start()
    for i in range(C):
        if i not in waited_outs:
            out_copy(i).wait()


def kernel(x, hippocampus, neocortex):
    B, S, H = x.shape
    return pl.pallas_call(
        _dma_pipe,
        out_shape=jax.ShapeDtypeStruct(x.shape, x.dtype),
        in_specs=[pl.BlockSpec(memory_space=pl.ANY)],
        out_specs=pl.BlockSpec(memory_space=pl.ANY),
        scratch_shapes=[
            pltpu.VMEM((_NBUF, _CHUNK_B, S, H), x.dtype),
            pltpu.SemaphoreType.DMA((_NBUF,)),
            pltpu.SemaphoreType.DMA((_NBUF,)),
        ],
    )(x)
